# baseline (speedup 1.0000x reference)
import functools

import jax
import jax.numpy as jnp
from jax import lax
from jax.experimental import pallas as pl
from jax.experimental.pallas import tpu as pltpu
from jax.experimental.pallas import tpu_sc as plsc

D_MODEL = 64
EPS = 1e-05

NUM_CORES = 2
NUM_SUBCORES = 16
NUM_WORKERS = NUM_CORES * NUM_SUBCORES

_DN = lax.GatherDimensionNumbers(
    offset_dims=(), collapsed_slice_dims=(0,), start_index_map=(0,))


def _lane_perm(x, idx):
    return lax.gather(x, idx[:, None], _DN, slice_sizes=(1,),
                      mode=lax.GatherScatterMode.PROMISE_IN_BOUNDS)


def _lane_allsum(x):
    lanes = lax.iota(jnp.int32, 16)
    for sh in (8, 4, 2, 1):
        x = x + _lane_perm(x, lanes ^ sh)
    return x


def _transpose16(regs, lanes):
    out = list(regs)
    s = 1
    while s < 16:
        nxt = list(out)
        m = (lanes & s) != 0
        down = (lanes - s) & 15
        up = (lanes + s) & 15
        for i in range(16):
            if i & s:
                continue
            a, b = out[i], out[i + s]
            nxt[i] = jnp.where(m, _lane_perm(b, down), a)
            nxt[i + s] = jnp.where(m, b, _lane_perm(a, up))
        out = nxt
        s *= 2
    return out


def _repack_table(emb_t, tail_emb, vocab):
    n_blk = vocab // 128 + (1 if vocab % 128 else 0)
    n_main = n_blk - 1
    per_w = (n_blk + NUM_WORKERS - 1) // NUM_WORKERS
    mesh = plsc.VectorSubcoreMesh(core_axis_name="c", subcore_axis_name="s")

    @functools.partial(
        pl.kernel,
        mesh=mesh,
        out_type=jax.ShapeDtypeStruct((vocab, 128), jnp.float32),
        scratch_types=[
            pltpu.VMEM((2, D_MODEL, 128), jnp.float32),
            pltpu.VMEM((2, 128, 128), jnp.float32),
            pltpu.VMEM((D_MODEL, 64), jnp.float32),
            pltpu.VMEM((D_MODEL, 128), jnp.float32),
            pltpu.SemaphoreType.DMA,
            pltpu.SemaphoreType.DMA,
        ],
        compiler_params=pltpu.CompilerParams(needs_layout_passes=False),
    )
    def body(emb_hbm, tail_hbm, out_hbm, in_v, tr_v, tail_v, tail_o,
             isem, osem):
        wid = lax.axis_index("s") * NUM_CORES + lax.axis_index("c")
        lanes = lax.iota(jnp.int32, 16)

        def in_copy(blk, buf):
            v0 = pl.multiple_of(blk * 128, 128)
            return pltpu.make_async_copy(
                emb_hbm.at[:, pl.ds(v0, 128)], in_v.at[buf], isem)

        def out_copy(blk, buf):
            v0 = pl.multiple_of(blk * 128, 128)
            return pltpu.make_async_copy(
                tr_v.at[buf], out_hbm.at[pl.ds(v0, 128)], osem)

        def transpose_block(buf, n_vt):
            def vt_step(vt, c1):
                for ct in range(4):
                    regs = [in_v[buf, ct * 16 + i, pl.ds(vt * 16, 16)]
                            for i in range(16)]
                    tr = _transpose16(regs, lanes)
                    for i in range(16):
                        tr_v[buf, vt * 16 + i, pl.ds(ct * 16, 16)] = tr[i]
                return c1
            lax.fori_loop(0, n_vt, vt_step, 0)

        @pl.when(wid < n_main)
        def _():
            in_copy(wid, 0).start()

        def step(t, carry):
            blk = wid + t * NUM_WORKERS
            buf = t % 2

            @pl.when(blk < n_main)
            def _():
                nxt = blk + NUM_WORKERS

                @pl.when(nxt < n_main)
                def _():
                    in_copy(nxt, (t + 1) % 2).start()
                in_copy(blk, buf).wait()

                @pl.when(t >= 2)
                def _():
                    out_copy(blk - 2 * NUM_WORKERS, buf).wait()
                transpose_block(buf, 8)
                out_copy(blk, buf).start()

            @pl.when(blk == n_blk - 1)
            def _():
                v0 = pl.multiple_of((n_blk - 1) * 128, 128)
                pltpu.sync_copy(tail_hbm, tail_v)

                def tail_row(i, c1):
                    for k in range(4):
                        tail_o[i, pl.ds(16 * k, 16)] = (
                            tail_v[i, pl.ds(16 * k, 16)])
                    return c1
                lax.fori_loop(0, 64, tail_row, 0)
                pltpu.sync_copy(tail_o, out_hbm.at[pl.ds(v0, 64)])
            return carry

        lax.fori_loop(0, per_w, step, 0)

        n_mine = (n_main - wid + NUM_WORKERS - 1) // NUM_WORKERS

        @pl.when(n_mine >= 1)
        def _():
            t_last = n_mine - 1
            out_copy(wid + t_last * NUM_WORKERS, t_last % 2).wait()

        @pl.when(n_mine >= 2)
        def _():
            t_prev = n_mine - 2
            out_copy(wid + t_prev * NUM_WORKERS, t_prev % 2).wait()

    return body(emb_t, tail_emb)


CHUNK2 = 128
NG = CHUNK2 // 128


def _gather_logmap(idx2d, pad_tab, n_rows):
    per_w = n_rows // NUM_WORKERS
    steps = per_w // CHUNK2
    mesh = plsc.VectorSubcoreMesh(core_axis_name="c", subcore_axis_name="s")

    @functools.partial(
        pl.kernel,
        mesh=mesh,
        out_type=jax.ShapeDtypeStruct((n_rows, D_MODEL), jnp.float32),
        scratch_types=[
            pltpu.VMEM((4, NG, 128), jnp.int32),
            pltpu.VMEM((4, CHUNK2, 128), jnp.float32),
            pltpu.VMEM((2, CHUNK2, D_MODEL), jnp.float32),
            pltpu.SemaphoreType.DMA,
            pltpu.SemaphoreType.DMA,
            pltpu.SemaphoreType.DMA,
        ],
        compiler_params=pltpu.CompilerParams(needs_layout_passes=False),
    )
    def body(idx_hbm, tab_hbm, out_hbm, idx_v, rows_v, obuf_v,
             isem, gsem, osem):
        wid = lax.axis_index("s") * NUM_CORES + lax.axis_index("c")
        row_base = wid * per_w
        irow_base = row_base // 128

        def idx_copy(g):
            off = pl.multiple_of(irow_base + g * NG, NG)
            return pltpu.make_async_copy(
                idx_hbm.at[pl.ds(off, NG)], idx_v.at[g % 4], isem)

        def gather_copy(g, b):
            return pltpu.make_async_copy(
                tab_hbm.at[idx_v.at[g % 4, b]],
                rows_v.at[g % 4, pl.ds(b * 128, 128)], gsem)

        def gather_wait(g, b):
            return pltpu.make_async_copy(
                tab_hbm.at[pl.ds(0, 128)],
                rows_v.at[g % 4, pl.ds(b * 128, 128)], gsem)

        def out_copy(g, buf):
            off = pl.multiple_of(row_base + g * CHUNK2, CHUNK2)
            return pltpu.make_async_copy(
                obuf_v.at[buf], out_hbm.at[pl.ds(off, CHUNK2)], osem)

        idx_copy(0).start()
        idx_copy(1).start()
        idx_copy(2).start()
        for j in range(2):
            idx_copy(j).wait()
            for b in range(NG):
                gather_copy(j, b).start()

        def step(g, carry):
            buf = g % 2

            @pl.when(g + 2 < steps)
            def _():
                idx_copy(g + 2).wait()
                for b in range(NG):
                    gather_copy(g + 2, b).start()

            for b in range(NG):
                gather_wait(g, b).wait()

            @pl.when(g + 3 < steps)
            def _():
                idx_copy(g + 3).start()

            rview = rows_v.at[g % 4]
            oview = obuf_v.at[buf]

            def row_fix(i, c):
                q = [rview[i, pl.ds(16 * k, 16)] for k in range(4)]
                s = (q[0] * q[0] + q[1] * q[1]) + (q[2] * q[2] + q[3] * q[3])
                nsv = _lane_allsum(s)
                scale = 2.0 / ((1.0 + EPS) - nsv)
                for k in range(4):
                    oview[i, pl.ds(16 * k, 16)] = q[k] * scale
                return c

            lax.fori_loop(0, CHUNK2, row_fix, 0, unroll=16)

            @pl.when(g >= 2)
            def _():
                out_copy(g - 2, buf).wait()
            out_copy(g, buf).start()
            return carry

        lax.fori_loop(0, steps, step, 0)
        out_copy(steps - 2, (steps - 2) % 2).wait()
        out_copy(steps - 1, (steps - 1) % 2).wait()

    return body(idx2d, pad_tab)


def kernel(token_ids, embeddings):
    bsz, seq = token_ids.shape
    vocab, d = embeddings.shape
    n_rows = bsz * seq
    n_full = (vocab // 128) * 128
    pad_tab = _repack_table(embeddings.T, embeddings[n_full:, :], vocab)
    idx2d = token_ids.reshape(n_rows // 128, 128).astype(jnp.int32)
    out = _gather_logmap(idx2d, pad_tab, n_rows)
    return out.reshape(bsz, seq, d)
